# baseline (speedup 1.0000x reference)
import functools

import jax
import jax.numpy as jnp
import numpy as np
from jax import lax
from jax.experimental import pallas as pl
from jax.experimental.pallas import tpu as pltpu
from jax.experimental.pallas import tpu_sc as plsc

_N = 10000
_E = 320000
_D = 128
_H = 4
_C = 32
_NC = 2
_NS = 16
_NW = _NC * _NS
_EPW = _E // _NW
_K = 80
_NCHUNK = _EPW // _K
_RPT = 624
_REM = _N - _NS * _RPT
_IB = 5
_IBK = _IB * _K
_NPDA = 320
_PD_T = 32
_PD_NT = _NPDA // _PD_T


def _mm_body(x_ref, wl_ref, wr_ref, xl_ref, xr_ref):
    xv = x_ref[...]
    xl_ref[...] = jnp.dot(xv, wl_ref[...], preferred_element_type=jnp.float32)
    xr_ref[...] = jnp.dot(xv, wr_ref[...], preferred_element_type=jnp.float32)


def _edge_body(xl_hbm, xr_hbm, src_hbm, dst_hbm, attf_hbm, z128_hbm,
              acc_out, den_out,
              srcblk, dstblk, dstv0, dstv1, dst8v0, dst8v1, xlr, xrr, att_v,
              accS, denP, gsem, ssem):
    cid = lax.axis_index("c")
    sid = lax.axis_index("s")
    wid = sid * _NC + cid
    base = wid * _EPW

    pltpu.sync_copy(attf_hbm, att_v)

    rows = pl.ds(sid * _RPT, _RPT)
    prows = pl.ds(sid * _PD_T, _PD_T)
    pltpu.sync_copy(z128_hbm.at[pl.ds(0, _RPT)], accS.at[rows])

    @pl.when(sid < _PD_NT)
    def _zero_den():
        pltpu.sync_copy(z128_hbm.at[pl.ds(0, _PD_T)], denP.at[prows])
    if _REM:
        rem = pl.ds(_NS * _RPT, _REM)

        @pl.when(sid == _NS - 1)
        def _zero_rem():
            pltpu.sync_copy(z128_hbm.at[pl.ds(0, _REM)], accS.at[rem])

    pltpu.sync_copy(src_hbm.at[pl.ds(base, _IBK)], srcblk)
    pltpu.sync_copy(dst_hbm.at[pl.ds(base, _IBK)], dstblk)
    pltpu.async_copy(xl_hbm.at[srcblk.at[pl.ds(0, _K)]],
                     xlr.at[pl.ds(0, _K)], gsem.at[0])
    pltpu.async_copy(xr_hbm.at[dstblk.at[pl.ds(0, _K)]],
                     xrr.at[pl.ds(0, _K)], gsem.at[0])

    plsc.subcore_barrier()

    iota16 = lax.iota(jnp.int32, 16)
    zz = jnp.zeros((16,), jnp.float32)

    def _chunk(ci, carry):
        b = ci & 1
        j5 = ci // _IB
        ioff = (ci - j5 * _IB) * _K
        boff = b * _K

        pltpu.make_async_copy(
            xl_hbm.at[srcblk.at[pl.ds(ioff, _K)]],
            xlr.at[pl.ds(boff, _K)], gsem.at[b]).wait()
        pltpu.make_async_copy(
            xr_hbm.at[dstblk.at[pl.ds(ioff, _K)]],
            xrr.at[pl.ds(boff, _K)], gsem.at[b]).wait()

        @pl.when((ci > 0) & (b == 1))
        def _w2a():
            pltpu.make_async_copy(
                xlr.at[pl.ds(0, _K)], accS.at[dstv0], ssem.at[0]).wait()
            pltpu.make_async_copy(
                xrr.at[pl.ds(0, _K)], denP.at[dst8v0], ssem.at[0]).wait()

        @pl.when((ci > 0) & (b == 0))
        def _w2b():
            pltpu.make_async_copy(
                xlr.at[pl.ds(_K, _K)], accS.at[dstv1], ssem.at[1]).wait()
            pltpu.make_async_copy(
                xrr.at[pl.ds(_K, _K)], denP.at[dst8v1], ssem.at[1]).wait()

        bs16 = jnp.full((16,), boff, jnp.int32)
        is16 = jnp.full((16,), ioff, jnp.int32)
        m0 = bs16 == 0
        m1 = jnp.logical_not(m0)

        def _pre(g, c2):
            eids = iota16 + (g * 16)
            dv = plsc.load_gather(dstblk, [eids + is16])
            plsc.store_scatter(dstv0, [eids], dv, mask=m0)
            plsc.store_scatter(dstv1, [eids], dv, mask=m1)
            d8 = lax.shift_right_logical(dv, 5)
            plsc.store_scatter(dst8v0, [eids], d8, mask=m0)
            plsc.store_scatter(dst8v1, [eids], d8, mask=m1)
            return c2
        lax.fori_loop(0, _K // 16, _pre, 0)

        nb = ci + 1

        @pl.when((nb % _IB == 0) & (nb < _NCHUNK))
        def _i1():
            pltpu.sync_copy(src_hbm.at[pl.ds(base + nb * _K, _IBK)], srcblk)
            pltpu.sync_copy(dst_hbm.at[pl.ds(base + nb * _K, _IBK)], dstblk)

        @pl.when(nb < _NCHUNK)
        def _i2():
            j5n = nb // _IB
            ioffn = (nb - j5n * _IB) * _K
            bn = 1 - b
            pltpu.async_copy(xl_hbm.at[srcblk.at[pl.ds(ioffn, _K)]],
                             xlr.at[pl.ds(bn * _K, _K)], gsem.at[bn])
            pltpu.async_copy(xr_hbm.at[dstblk.at[pl.ds(ioffn, _K)]],
                             xrr.at[pl.ds(bn * _K, _K)], gsem.at[bn])


        def _group(g, c2):
            eids = iota16 + (g * 16)
            ebs = eids + bs16

            def _haccum(h):
                def _hbody(i, carry):
                    fvc, a = carry
                    for j in range(8):
                        lv = plsc.load_gather(xlr, [ebs, fvc])
                        rv = plsc.load_gather(xrr, [ebs, fvc])
                        av = plsc.load_gather(att_v, [fvc])
                        t = lv + rv
                        t = jnp.maximum(t, 0.2 * t)
                        a = a + t * av
                        fvc = fvc + 1
                    return (fvc, a)
                _, a = lax.fori_loop(
                    0, _C // 8, _hbody,
                    (jnp.full((16,), h * _C, jnp.int32),
                     jnp.zeros((16,), jnp.float32)))
                return a

            acc = [_haccum(h) for h in range(_H)]
            p = [jnp.exp(acc[h]) for h in range(_H)]
            dv = jnp.where(m0, plsc.load_gather(dstv0, [eids]),
                           plsc.load_gather(dstv1, [eids]))
            cv0 = (dv & 31) * 4
            for h in range(_H):
                def _sbody(i, fvc):
                    for j in range(8):
                        lv = plsc.load_gather(xlr, [ebs, fvc])
                        plsc.store_scatter(xlr, [ebs, fvc], lv * p[h])
                        plsc.store_scatter(xrr, [ebs, fvc], zz)
                        fvc = fvc + 1
                    return fvc
                lax.fori_loop(0, _C // 8, _sbody,
                              jnp.full((16,), h * _C, jnp.int32))
            for h in range(_H):
                plsc.store_scatter(xrr, [ebs, cv0 + h], p[h])
            return c2
        lax.fori_loop(0, _K // 16, _group, 0)

        @pl.when(b == 0)
        def _s0():
            pltpu.async_copy(xlr.at[pl.ds(0, _K)], accS.at[dstv0],
                             ssem.at[0], add=True)
            pltpu.async_copy(xrr.at[pl.ds(0, _K)], denP.at[dst8v0],
                             ssem.at[0], add=True)

        @pl.when(b == 1)
        def _s1():
            pltpu.async_copy(xlr.at[pl.ds(_K, _K)], accS.at[dstv1],
                             ssem.at[1], add=True)
            pltpu.async_copy(xrr.at[pl.ds(_K, _K)], denP.at[dst8v1],
                             ssem.at[1], add=True)
        return carry
    lax.fori_loop(0, _NCHUNK, _chunk, 0)

    bL = (_NCHUNK - 1) & 1
    if bL == 0:
        pltpu.make_async_copy(xlr.at[pl.ds(0, _K)], accS.at[dstv0],
                              ssem.at[0]).wait()
        pltpu.make_async_copy(xrr.at[pl.ds(0, _K)], denP.at[dst8v0],
                              ssem.at[0]).wait()
    else:
        pltpu.make_async_copy(xlr.at[pl.ds(_K, _K)], accS.at[dstv1],
                              ssem.at[1]).wait()
        pltpu.make_async_copy(xrr.at[pl.ds(_K, _K)], denP.at[dst8v1],
                              ssem.at[1]).wait()

    plsc.subcore_barrier()
    pltpu.sync_copy(accS.at[rows], acc_out.at[cid, rows])

    @pl.when(sid < _PD_NT)
    def _copy_den():
        pltpu.sync_copy(denP.at[prows], den_out.at[cid, prows])
    if _REM:
        rem = pl.ds(_NS * _RPT, _REM)

        @pl.when(sid == _NS - 1)
        def _copy_rem():
            pltpu.sync_copy(accS.at[rem], acc_out.at[cid, rem])


_edge_kernel = functools.partial(
    pl.kernel,
    out_type=[
        jax.ShapeDtypeStruct((_NC, _N, _D), jnp.float32),
        jax.ShapeDtypeStruct((_NC, _NPDA, _D), jnp.float32),
    ],
    mesh=plsc.VectorSubcoreMesh(core_axis_name="c", subcore_axis_name="s"),
    compiler_params=pltpu.CompilerParams(needs_layout_passes=False),
    scratch_types=[
        pltpu.VMEM((_IBK,), jnp.int32),
        pltpu.VMEM((_IBK,), jnp.int32),
        pltpu.VMEM((_K,), jnp.int32),
        pltpu.VMEM((_K,), jnp.int32),
        pltpu.VMEM((_K,), jnp.int32),
        pltpu.VMEM((_K,), jnp.int32),
        pltpu.VMEM((2 * _K, _D), jnp.float32),
        pltpu.VMEM((2 * _K, _D), jnp.float32),
        pltpu.VMEM((_D,), jnp.float32),
        pltpu.VMEM_SHARED((_N, _D), jnp.float32),
        pltpu.VMEM_SHARED((_NPDA, _D), jnp.float32),
        pltpu.SemaphoreType.DMA((2,)),
        pltpu.SemaphoreType.DMA((2,)),
    ],
)(_edge_body)


def _post_body(x_ref, xl_ref, xr_ref, acc_ref, den_ref, attf_ref, bias_ref,
               gnw_ref, gnb_ref, gms_ref, out_ref):
    xv = x_ref[...]
    xl = xl_ref[...]
    xr = xr_ref[...]
    t = xl + xr
    t = jnp.maximum(t, 0.2 * t)
    w = t * attf_ref[...]
    ii = lax.broadcasted_iota(jnp.int32, (_D, _H), 0) // _C
    hh = lax.broadcasted_iota(jnp.int32, (_D, _H), 1)
    sel = (ii == hh).astype(jnp.float32)
    logit_s = jnp.dot(w, sel, preferred_element_type=jnp.float32)
    p_s = jnp.exp(logit_s)
    expand = jnp.dot(p_s, sel.T, preferred_element_type=jnp.float32)
    num = acc_ref[0] + acc_ref[1] + expand * xl
    den_e = jnp.dot(den_ref[0] + den_ref[1], sel.T,
                    preferred_element_type=jnp.float32) + expand
    h = num / (den_e + 1e-16) + bias_ref[...] + xv
    mean = jnp.mean(h, axis=0, keepdims=True)
    o = h - mean * gms_ref[...]
    var = jnp.mean(o * o, axis=0, keepdims=True)
    g = gnw_ref[...] * o / jnp.sqrt(var + 1e-5) + gnb_ref[...]
    out_ref[...] = 0.5 * g * (1.0 + lax.erf(g * np.float32(1.0 / np.sqrt(2.0))))


def kernel(x, edge_index, W_l, W_r, att, bias, gn_weight, gn_bias,
           gn_mean_scale):
    src = edge_index[0].astype(jnp.int32)
    dst = edge_index[1].astype(jnp.int32)
    attf = att.reshape(_H * _C).astype(jnp.float32)
    xl, xr = pl.pallas_call(
        _mm_body,
        out_shape=[jax.ShapeDtypeStruct((_N, _D), jnp.float32)] * 2,
    )(x, W_l, W_r)
    z128 = jnp.zeros((_RPT, _D), jnp.float32)
    acc, den_pack = _edge_kernel(xl, xr, src, dst, attf, z128)
    den4 = den_pack.reshape(_NC, _NPDA * 32, _H)[:, :_N]
    out = pl.pallas_call(
        _post_body,
        out_shape=jax.ShapeDtypeStruct((_N, _D), jnp.float32),
    )(x, xl, xr, acc, den4, attf.reshape(1, _D), bias.reshape(1, _D),
      gn_weight.reshape(1, _D), gn_bias.reshape(1, _D),
      gn_mean_scale.reshape(1, _D))
    return out


# speedup vs baseline: 5.2557x; 5.2557x over previous
import functools

import jax
import jax.numpy as jnp
import numpy as np
from jax import lax
from jax.experimental import pallas as pl
from jax.experimental.pallas import tpu as pltpu
from jax.experimental.pallas import tpu_sc as plsc

_N = 10000
_E = 320000
_D = 128
_H = 4
_C = 32
_NC = 2
_NS = 16
_NW = _NC * _NS
_EPW = _E // _NW
_K = 80
_NCHUNK = _EPW // _K
_RPT = 624
_REM = _N - _NS * _RPT
_IB = 5
_IBK = _IB * _K
_NPDA = 320
_PD_T = 32
_PD_NT = _NPDA // _PD_T


def _mm_body(x_ref, wl_ref, wr_ref, xl_ref, xr_ref):
    xv = x_ref[...]
    xl_ref[...] = jnp.dot(xv, wl_ref[...], preferred_element_type=jnp.float32)
    xr_ref[...] = jnp.dot(xv, wr_ref[...], preferred_element_type=jnp.float32)


def _edge_body(xl_hbm, xr_hbm, src_hbm, dst_hbm, attf_hbm, z128_hbm,
              acc_out, den_out,
              srcblk, dstblk, dstv0, dstv1, dst8v0, dst8v1, cbv, wov,
              xlr, xrr, att_v, accS, denP, gsem, ssem):
    cid = lax.axis_index("c")
    sid = lax.axis_index("s")
    wid = sid * _NC + cid
    base = wid * _EPW

    pltpu.sync_copy(attf_hbm, att_v)

    rows = pl.ds(sid * _RPT, _RPT)
    prows = pl.ds(sid * _PD_T, _PD_T)
    pltpu.sync_copy(z128_hbm.at[pl.ds(0, _RPT)], accS.at[rows])

    @pl.when(sid < _PD_NT)
    def _zero_den():
        pltpu.sync_copy(z128_hbm.at[pl.ds(0, _PD_T)], denP.at[prows])
    if _REM:
        rem = pl.ds(_NS * _RPT, _REM)

        @pl.when(sid == _NS - 1)
        def _zero_rem():
            pltpu.sync_copy(z128_hbm.at[pl.ds(0, _REM)], accS.at[rem])

    pltpu.sync_copy(src_hbm.at[pl.ds(base, _IBK)], srcblk)
    pltpu.sync_copy(dst_hbm.at[pl.ds(base, _IBK)], dstblk)
    pltpu.async_copy(xl_hbm.at[srcblk.at[pl.ds(0, _K)]],
                     xlr.at[pl.ds(0, _K)], gsem.at[0])
    pltpu.async_copy(xr_hbm.at[dstblk.at[pl.ds(0, _K)]],
                     xrr.at[pl.ds(0, _K)], gsem.at[0])

    plsc.subcore_barrier()

    iota16 = lax.iota(jnp.int32, 16)
    zz = jnp.zeros((16,), jnp.float32)

    def _chunk(ci, carry):
        b = ci & 1
        j5 = ci // _IB
        ioff = (ci - j5 * _IB) * _K
        boff = b * _K

        pltpu.make_async_copy(
            xl_hbm.at[srcblk.at[pl.ds(ioff, _K)]],
            xlr.at[pl.ds(boff, _K)], gsem.at[b]).wait()
        pltpu.make_async_copy(
            xr_hbm.at[dstblk.at[pl.ds(ioff, _K)]],
            xrr.at[pl.ds(boff, _K)], gsem.at[b]).wait()

        @pl.when((ci > 0) & (b == 1))
        def _w2a():
            pltpu.make_async_copy(
                xlr.at[pl.ds(0, _K)], accS.at[dstv0], ssem.at[0]).wait()
            pltpu.make_async_copy(
                xrr.at[pl.ds(0, _K)], denP.at[dst8v0], ssem.at[0]).wait()

        @pl.when((ci > 0) & (b == 0))
        def _w2b():
            pltpu.make_async_copy(
                xlr.at[pl.ds(_K, _K)], accS.at[dstv1], ssem.at[1]).wait()
            pltpu.make_async_copy(
                xrr.at[pl.ds(_K, _K)], denP.at[dst8v1], ssem.at[1]).wait()

        bs16 = jnp.full((16,), boff, jnp.int32)
        is16 = jnp.full((16,), ioff, jnp.int32)
        m0 = bs16 == 0
        m1 = jnp.logical_not(m0)

        def _pre(g, c2):
            eids = iota16 + (g * 16)
            dv = plsc.load_gather(dstblk, [eids + is16])
            plsc.store_scatter(dstv0, [eids], dv, mask=m0)
            plsc.store_scatter(dstv1, [eids], dv, mask=m1)
            d8 = lax.shift_right_logical(dv, 5)
            plsc.store_scatter(dst8v0, [eids], d8, mask=m0)
            plsc.store_scatter(dst8v1, [eids], d8, mask=m1)
            dm = dv & 31
            plsc.store_scatter(cbv, [eids],
                               lax.shift_left(lax.shift_right_logical(dm, 2),
                                              4))
            plsc.store_scatter(wov, [eids], (dm & 3) * 4)
            return c2
        lax.fori_loop(0, _K // 16, _pre, 0)

        nb = ci + 1

        @pl.when((nb % _IB == 0) & (nb < _NCHUNK))
        def _i1():
            pltpu.sync_copy(src_hbm.at[pl.ds(base + nb * _K, _IBK)], srcblk)
            pltpu.sync_copy(dst_hbm.at[pl.ds(base + nb * _K, _IBK)], dstblk)

        @pl.when(nb < _NCHUNK)
        def _i2():
            j5n = nb // _IB
            ioffn = (nb - j5n * _IB) * _K
            bn = 1 - b
            pltpu.async_copy(xl_hbm.at[srcblk.at[pl.ds(ioffn, _K)]],
                             xlr.at[pl.ds(bn * _K, _K)], gsem.at[bn])
            pltpu.async_copy(xr_hbm.at[dstblk.at[pl.ds(ioffn, _K)]],
                             xrr.at[pl.ds(bn * _K, _K)], gsem.at[bn])


        att8 = [att_v[pl.ds(16 * bk, 16)] for bk in range(_D // 16)]

        def _edge(e, c2):
            row = boff + e
            xlv = [xlr[row, pl.ds(16 * bk, 16)] for bk in range(_D // 16)]
            w = []
            for bk in range(_D // 16):
                t = xlv[bk] + xrr[row, pl.ds(16 * bk, 16)]
                t = jnp.maximum(t, 0.2 * t)
                w.append(t * att8[bk])
            pv = []
            for h in range(_H):
                sh = w[2 * h] + w[2 * h + 1]
                r = jnp.sum(sh)
                pv.append(jnp.exp(jnp.full((16,), r, jnp.float32)))
            for bk in range(_D // 16):
                xlr[row, pl.ds(16 * bk, 16)] = xlv[bk] * pv[bk // 2]
            es = jnp.full((16,), e, jnp.int32)
            cb16 = plsc.load_gather(cbv, [es])[0]
            l4 = iota16 - plsc.load_gather(wov, [es])
            pd = jnp.where(l4 == 0, pv[0],
                           jnp.where(l4 == 1, pv[1],
                                     jnp.where(l4 == 2, pv[2],
                                               jnp.where(l4 == 3, pv[3],
                                                         zz))))
            for bk in range(_D // 16):
                xrr[row, pl.ds(16 * bk, 16)] = zz
            xrr[row, pl.ds(cb16, 16)] = pd
            return c2
        lax.fori_loop(0, _K, _edge, 0)

        @pl.when(b == 0)
        def _s0():
            pltpu.async_copy(xlr.at[pl.ds(0, _K)], accS.at[dstv0],
                             ssem.at[0], add=True)
            pltpu.async_copy(xrr.at[pl.ds(0, _K)], denP.at[dst8v0],
                             ssem.at[0], add=True)

        @pl.when(b == 1)
        def _s1():
            pltpu.async_copy(xlr.at[pl.ds(_K, _K)], accS.at[dstv1],
                             ssem.at[1], add=True)
            pltpu.async_copy(xrr.at[pl.ds(_K, _K)], denP.at[dst8v1],
                             ssem.at[1], add=True)
        return carry
    lax.fori_loop(0, _NCHUNK, _chunk, 0)

    bL = (_NCHUNK - 1) & 1
    if bL == 0:
        pltpu.make_async_copy(xlr.at[pl.ds(0, _K)], accS.at[dstv0],
                              ssem.at[0]).wait()
        pltpu.make_async_copy(xrr.at[pl.ds(0, _K)], denP.at[dst8v0],
                              ssem.at[0]).wait()
    else:
        pltpu.make_async_copy(xlr.at[pl.ds(_K, _K)], accS.at[dstv1],
                              ssem.at[1]).wait()
        pltpu.make_async_copy(xrr.at[pl.ds(_K, _K)], denP.at[dst8v1],
                              ssem.at[1]).wait()

    plsc.subcore_barrier()
    pltpu.sync_copy(accS.at[rows], acc_out.at[cid, rows])

    @pl.when(sid < _PD_NT)
    def _copy_den():
        pltpu.sync_copy(denP.at[prows], den_out.at[cid, prows])
    if _REM:
        rem = pl.ds(_NS * _RPT, _REM)

        @pl.when(sid == _NS - 1)
        def _copy_rem():
            pltpu.sync_copy(accS.at[rem], acc_out.at[cid, rem])


_edge_kernel = functools.partial(
    pl.kernel,
    out_type=[
        jax.ShapeDtypeStruct((_NC, _N, _D), jnp.float32),
        jax.ShapeDtypeStruct((_NC, _NPDA, _D), jnp.float32),
    ],
    mesh=plsc.VectorSubcoreMesh(core_axis_name="c", subcore_axis_name="s"),
    compiler_params=pltpu.CompilerParams(needs_layout_passes=False),
    scratch_types=[
        pltpu.VMEM((_IBK,), jnp.int32),
        pltpu.VMEM((_IBK,), jnp.int32),
        pltpu.VMEM((_K,), jnp.int32),
        pltpu.VMEM((_K,), jnp.int32),
        pltpu.VMEM((_K,), jnp.int32),
        pltpu.VMEM((_K,), jnp.int32),
        pltpu.VMEM((_K,), jnp.int32),
        pltpu.VMEM((_K,), jnp.int32),
        pltpu.VMEM((2 * _K, _D), jnp.float32),
        pltpu.VMEM((2 * _K, _D), jnp.float32),
        pltpu.VMEM((_D,), jnp.float32),
        pltpu.VMEM_SHARED((_N, _D), jnp.float32),
        pltpu.VMEM_SHARED((_NPDA, _D), jnp.float32),
        pltpu.SemaphoreType.DMA((2,)),
        pltpu.SemaphoreType.DMA((2,)),
    ],
)(_edge_body)


def _post_body(x_ref, xl_ref, xr_ref, acc_ref, den_ref, attf_ref, bias_ref,
               gnw_ref, gnb_ref, gms_ref, out_ref):
    xv = x_ref[...]
    xl = xl_ref[...]
    xr = xr_ref[...]
    t = xl + xr
    t = jnp.maximum(t, 0.2 * t)
    w = t * attf_ref[...]
    ii = lax.broadcasted_iota(jnp.int32, (_D, _H), 0) // _C
    hh = lax.broadcasted_iota(jnp.int32, (_D, _H), 1)
    sel = (ii == hh).astype(jnp.float32)
    logit_s = jnp.dot(w, sel, preferred_element_type=jnp.float32)
    p_s = jnp.exp(logit_s)
    expand = jnp.dot(p_s, sel.T, preferred_element_type=jnp.float32)
    num = acc_ref[0] + acc_ref[1] + expand * xl
    den_e = jnp.dot(den_ref[0] + den_ref[1], sel.T,
                    preferred_element_type=jnp.float32) + expand
    h = num / (den_e + 1e-16) + bias_ref[...] + xv
    mean = jnp.mean(h, axis=0, keepdims=True)
    o = h - mean * gms_ref[...]
    var = jnp.mean(o * o, axis=0, keepdims=True)
    g = gnw_ref[...] * o / jnp.sqrt(var + 1e-5) + gnb_ref[...]
    out_ref[...] = 0.5 * g * (1.0 + lax.erf(g * np.float32(1.0 / np.sqrt(2.0))))


def kernel(x, edge_index, W_l, W_r, att, bias, gn_weight, gn_bias,
           gn_mean_scale):
    src = edge_index[0].astype(jnp.int32)
    dst = edge_index[1].astype(jnp.int32)
    attf = att.reshape(_H * _C).astype(jnp.float32)
    xl, xr = pl.pallas_call(
        _mm_body,
        out_shape=[jax.ShapeDtypeStruct((_N, _D), jnp.float32)] * 2,
    )(x, W_l, W_r)
    z128 = jnp.zeros((_RPT, _D), jnp.float32)
    acc, den_pack = _edge_kernel(xl, xr, src, dst, attf, z128)
    den4 = den_pack.reshape(_NC, _NPDA * 32, _H)[:, :_N]
    out = pl.pallas_call(
        _post_body,
        out_shape=jax.ShapeDtypeStruct((_N, _D), jnp.float32),
    )(x, xl, xr, acc, den4, attf.reshape(1, _D), bias.reshape(1, _D),
      gn_weight.reshape(1, _D), gn_bias.reshape(1, _D),
      gn_mean_scale.reshape(1, _D))
    return out


# speedup vs baseline: 8.7860x; 1.6717x over previous
import functools

import jax
import jax.numpy as jnp
import numpy as np
from jax import lax
from jax.experimental import pallas as pl
from jax.experimental.pallas import tpu as pltpu
from jax.experimental.pallas import tpu_sc as plsc

_N = 10000
_E = 320000
_D = 128
_H = 4
_C = 32
_NC = 2
_NS = 16
_NW = _NC * _NS
_EPW = _E // _NW
_K = 80
_NCHUNK = _EPW // _K
_RPT = 624
_REM = _N - _NS * _RPT
_IB = 5
_IBK = _IB * _K
_NPDA = 320
_PD_T = 32
_PD_NT = _NPDA // _PD_T


def _mm_body(x_ref, wl_ref, wr_ref, xl_ref, xr_ref):
    xv = x_ref[...]
    xl_ref[...] = jnp.dot(xv, wl_ref[...], preferred_element_type=jnp.float32)
    xr_ref[...] = jnp.dot(xv, wr_ref[...], preferred_element_type=jnp.float32)


def _edge_body(xl_hbm, xr_hbm, src_hbm, dst_hbm, attf_hbm, z128_hbm,
              acc_out, den_out,
              srcblk, dstblk, dstv0, dstv1, dst8v0, dst8v1, cbv, wov,
              xlr, xrr, att_v, accS, denP, gsem, ssem):
    cid = lax.axis_index("c")
    sid = lax.axis_index("s")
    wid = sid * _NC + cid
    base = wid * _EPW

    pltpu.sync_copy(attf_hbm, att_v)

    rows = pl.ds(sid * _RPT, _RPT)
    prows = pl.ds(sid * _PD_T, _PD_T)
    pltpu.sync_copy(z128_hbm.at[pl.ds(0, _RPT)], accS.at[rows])

    @pl.when(sid < _PD_NT)
    def _zero_den():
        pltpu.sync_copy(z128_hbm.at[pl.ds(0, _PD_T)], denP.at[prows])
    if _REM:
        rem = pl.ds(_NS * _RPT, _REM)

        @pl.when(sid == _NS - 1)
        def _zero_rem():
            pltpu.sync_copy(z128_hbm.at[pl.ds(0, _REM)], accS.at[rem])

    pltpu.sync_copy(src_hbm.at[pl.ds(base, _IBK)], srcblk)
    pltpu.sync_copy(dst_hbm.at[pl.ds(base, _IBK)], dstblk)
    pltpu.async_copy(xl_hbm.at[srcblk.at[pl.ds(0, _K)]],
                     xlr.at[pl.ds(0, _K)], gsem.at[0])
    pltpu.async_copy(xr_hbm.at[dstblk.at[pl.ds(0, _K)]],
                     xrr.at[pl.ds(0, _K)], gsem.at[0])

    plsc.subcore_barrier()

    iota16 = lax.iota(jnp.int32, 16)
    zz = jnp.zeros((16,), jnp.float32)

    def _chunk(ci, carry):
        b = ci & 1
        j5 = ci // _IB
        ioff = (ci - j5 * _IB) * _K
        boff = b * _K

        pltpu.make_async_copy(
            xl_hbm.at[srcblk.at[pl.ds(ioff, _K)]],
            xlr.at[pl.ds(boff, _K)], gsem.at[b]).wait()
        pltpu.make_async_copy(
            xr_hbm.at[dstblk.at[pl.ds(ioff, _K)]],
            xrr.at[pl.ds(boff, _K)], gsem.at[b]).wait()

        @pl.when((ci > 0) & (b == 1))
        def _w2a():
            pltpu.make_async_copy(
                xlr.at[pl.ds(0, _K)], accS.at[dstv0], ssem.at[0]).wait()
            pltpu.make_async_copy(
                xrr.at[pl.ds(0, _K)], denP.at[dst8v0], ssem.at[0]).wait()

        @pl.when((ci > 0) & (b == 0))
        def _w2b():
            pltpu.make_async_copy(
                xlr.at[pl.ds(_K, _K)], accS.at[dstv1], ssem.at[1]).wait()
            pltpu.make_async_copy(
                xrr.at[pl.ds(_K, _K)], denP.at[dst8v1], ssem.at[1]).wait()

        bs16 = jnp.full((16,), boff, jnp.int32)
        is16 = jnp.full((16,), ioff, jnp.int32)
        m0 = bs16 == 0
        m1 = jnp.logical_not(m0)

        def _pre(g, c2):
            eids = iota16 + (g * 16)
            dv = plsc.load_gather(dstblk, [eids + is16])
            plsc.store_scatter(dstv0, [eids], dv, mask=m0)
            plsc.store_scatter(dstv1, [eids], dv, mask=m1)
            d8 = lax.shift_right_logical(dv, 5)
            plsc.store_scatter(dst8v0, [eids], d8, mask=m0)
            plsc.store_scatter(dst8v1, [eids], d8, mask=m1)
            dm = dv & 31
            plsc.store_scatter(cbv, [eids],
                               lax.shift_left(lax.shift_right_logical(dm, 2),
                                              4))
            plsc.store_scatter(wov, [eids], (dm & 3) * 4)
            return c2
        lax.fori_loop(0, _K // 16, _pre, 0)

        nb = ci + 1

        @pl.when((nb % _IB == 0) & (nb < _NCHUNK))
        def _i1():
            pltpu.sync_copy(src_hbm.at[pl.ds(base + nb * _K, _IBK)], srcblk)
            pltpu.sync_copy(dst_hbm.at[pl.ds(base + nb * _K, _IBK)], dstblk)

        @pl.when(nb < _NCHUNK)
        def _i2():
            j5n = nb // _IB
            ioffn = (nb - j5n * _IB) * _K
            bn = 1 - b
            pltpu.async_copy(xl_hbm.at[srcblk.at[pl.ds(ioffn, _K)]],
                             xlr.at[pl.ds(bn * _K, _K)], gsem.at[bn])
            pltpu.async_copy(xr_hbm.at[dstblk.at[pl.ds(ioffn, _K)]],
                             xrr.at[pl.ds(bn * _K, _K)], gsem.at[bn])


        att8 = [att_v[pl.ds(16 * bk, 16)] for bk in range(_D // 16)]

        def _edge(e, c2):
            row = boff + e
            xlv = [xlr[row, pl.ds(16 * bk, 16)] for bk in range(_D // 16)]
            w = []
            for bk in range(_D // 16):
                t = xlv[bk] + xrr[row, pl.ds(16 * bk, 16)]
                t = jnp.maximum(t, 0.2 * t)
                w.append(t * att8[bk])
            pv = []
            for h in range(_H):
                sh = w[2 * h] + w[2 * h + 1]
                r = jnp.sum(sh)
                pv.append(jnp.exp(jnp.full((16,), r, jnp.float32)))
            for bk in range(_D // 16):
                xlr[row, pl.ds(16 * bk, 16)] = xlv[bk] * pv[bk // 2]
            es = jnp.full((16,), e, jnp.int32)
            cb16 = plsc.load_gather(cbv, [es])[0]
            l4 = iota16 - plsc.load_gather(wov, [es])
            pd = jnp.where(l4 == 0, pv[0],
                           jnp.where(l4 == 1, pv[1],
                                     jnp.where(l4 == 2, pv[2],
                                               jnp.where(l4 == 3, pv[3],
                                                         zz))))
            for bk in range(_D // 16):
                xrr[row, pl.ds(16 * bk, 16)] = zz
            xrr[row, pl.ds(cb16, 16)] = pd
            return c2

        def _edge_pl(e):
            _edge(e, 0)
        plsc.parallel_loop(0, _K, 1, unroll=2)(_edge_pl)

        @pl.when(b == 0)
        def _s0():
            pltpu.async_copy(xlr.at[pl.ds(0, _K)], accS.at[dstv0],
                             ssem.at[0], add=True)
            pltpu.async_copy(xrr.at[pl.ds(0, _K)], denP.at[dst8v0],
                             ssem.at[0], add=True)

        @pl.when(b == 1)
        def _s1():
            pltpu.async_copy(xlr.at[pl.ds(_K, _K)], accS.at[dstv1],
                             ssem.at[1], add=True)
            pltpu.async_copy(xrr.at[pl.ds(_K, _K)], denP.at[dst8v1],
                             ssem.at[1], add=True)
        return carry
    lax.fori_loop(0, _NCHUNK, _chunk, 0)

    bL = (_NCHUNK - 1) & 1
    if bL == 0:
        pltpu.make_async_copy(xlr.at[pl.ds(0, _K)], accS.at[dstv0],
                              ssem.at[0]).wait()
        pltpu.make_async_copy(xrr.at[pl.ds(0, _K)], denP.at[dst8v0],
                              ssem.at[0]).wait()
    else:
        pltpu.make_async_copy(xlr.at[pl.ds(_K, _K)], accS.at[dstv1],
                              ssem.at[1]).wait()
        pltpu.make_async_copy(xrr.at[pl.ds(_K, _K)], denP.at[dst8v1],
                              ssem.at[1]).wait()

    plsc.subcore_barrier()
    pltpu.sync_copy(accS.at[rows], acc_out.at[cid, rows])

    @pl.when(sid < _PD_NT)
    def _copy_den():
        pltpu.sync_copy(denP.at[prows], den_out.at[cid, prows])
    if _REM:
        rem = pl.ds(_NS * _RPT, _REM)

        @pl.when(sid == _NS - 1)
        def _copy_rem():
            pltpu.sync_copy(accS.at[rem], acc_out.at[cid, rem])


_edge_kernel = functools.partial(
    pl.kernel,
    out_type=[
        jax.ShapeDtypeStruct((_NC, _N, _D), jnp.float32),
        jax.ShapeDtypeStruct((_NC, _NPDA, _D), jnp.float32),
    ],
    mesh=plsc.VectorSubcoreMesh(core_axis_name="c", subcore_axis_name="s"),
    compiler_params=pltpu.CompilerParams(needs_layout_passes=False),
    scratch_types=[
        pltpu.VMEM((_IBK,), jnp.int32),
        pltpu.VMEM((_IBK,), jnp.int32),
        pltpu.VMEM((_K,), jnp.int32),
        pltpu.VMEM((_K,), jnp.int32),
        pltpu.VMEM((_K,), jnp.int32),
        pltpu.VMEM((_K,), jnp.int32),
        pltpu.VMEM((_K,), jnp.int32),
        pltpu.VMEM((_K,), jnp.int32),
        pltpu.VMEM((2 * _K, _D), jnp.float32),
        pltpu.VMEM((2 * _K, _D), jnp.float32),
        pltpu.VMEM((_D,), jnp.float32),
        pltpu.VMEM_SHARED((_N, _D), jnp.float32),
        pltpu.VMEM_SHARED((_NPDA, _D), jnp.float32),
        pltpu.SemaphoreType.DMA((2,)),
        pltpu.SemaphoreType.DMA((2,)),
    ],
)(_edge_body)


def _post_body(x_ref, xl_ref, xr_ref, acc_ref, den_ref, attf_ref, bias_ref,
               gnw_ref, gnb_ref, gms_ref, out_ref):
    xv = x_ref[...]
    xl = xl_ref[...]
    xr = xr_ref[...]
    t = xl + xr
    t = jnp.maximum(t, 0.2 * t)
    w = t * attf_ref[...]
    ii = lax.broadcasted_iota(jnp.int32, (_D, _H), 0) // _C
    hh = lax.broadcasted_iota(jnp.int32, (_D, _H), 1)
    sel = (ii == hh).astype(jnp.float32)
    logit_s = jnp.dot(w, sel, preferred_element_type=jnp.float32)
    p_s = jnp.exp(logit_s)
    expand = jnp.dot(p_s, sel.T, preferred_element_type=jnp.float32)
    num = acc_ref[0] + acc_ref[1] + expand * xl
    den_e = jnp.dot(den_ref[0] + den_ref[1], sel.T,
                    preferred_element_type=jnp.float32) + expand
    h = num / (den_e + 1e-16) + bias_ref[...] + xv
    mean = jnp.mean(h, axis=0, keepdims=True)
    o = h - mean * gms_ref[...]
    var = jnp.mean(o * o, axis=0, keepdims=True)
    g = gnw_ref[...] * o / jnp.sqrt(var + 1e-5) + gnb_ref[...]
    out_ref[...] = 0.5 * g * (1.0 + lax.erf(g * np.float32(1.0 / np.sqrt(2.0))))


def kernel(x, edge_index, W_l, W_r, att, bias, gn_weight, gn_bias,
           gn_mean_scale):
    src = edge_index[0].astype(jnp.int32)
    dst = edge_index[1].astype(jnp.int32)
    attf = att.reshape(_H * _C).astype(jnp.float32)
    xl, xr = pl.pallas_call(
        _mm_body,
        out_shape=[jax.ShapeDtypeStruct((_N, _D), jnp.float32)] * 2,
    )(x, W_l, W_r)
    z128 = jnp.zeros((_RPT, _D), jnp.float32)
    acc, den_pack = _edge_kernel(xl, xr, src, dst, attf, z128)
    den4 = den_pack.reshape(_NC, _NPDA * 32, _H)[:, :_N]
    out = pl.pallas_call(
        _post_body,
        out_shape=jax.ShapeDtypeStruct((_N, _D), jnp.float32),
    )(x, xl, xr, acc, den4, attf.reshape(1, _D), bias.reshape(1, _D),
      gn_weight.reshape(1, _D), gn_bias.reshape(1, _D),
      gn_mean_scale.reshape(1, _D))
    return out


# speedup vs baseline: 9.4464x; 1.0752x over previous
import functools

import jax
import jax.numpy as jnp
import numpy as np
from jax import lax
from jax.experimental import pallas as pl
from jax.experimental.pallas import tpu as pltpu
from jax.experimental.pallas import tpu_sc as plsc

_N = 10000
_E = 320000
_D = 128
_H = 4
_C = 32
_NC = 2
_NS = 16
_NW = _NC * _NS
_EPW = _E // _NW
_K = 80
_NCHUNK = _EPW // _K
_RPT = 624
_REM = _N - _NS * _RPT
_IB = 5
_IBK = _IB * _K
_NPDA = 320
_PD_T = 32
_PD_NT = _NPDA // _PD_T


def _mm_body(x_ref, wl_ref, wr_ref, xl_ref, xr_ref):
    xv = x_ref[...]
    xl_ref[...] = jnp.dot(xv, wl_ref[...], preferred_element_type=jnp.float32)
    xr_ref[...] = jnp.dot(xv, wr_ref[...], preferred_element_type=jnp.float32)


def _edge_body(xl_hbm, xr_hbm, src_hbm, dst_hbm, attf_hbm, z128_hbm,
              acc_out, den_out,
              srcblk, dstblk, dstv0, dstv1, dst8v0, dst8v1, cbv, wov,
              xlr, xrr, att_v, accS, denP, gsem, ssem):
    cid = lax.axis_index("c")
    sid = lax.axis_index("s")
    wid = sid * _NC + cid
    base = wid * _EPW

    pltpu.sync_copy(attf_hbm, att_v)

    rows = pl.ds(sid * _RPT, _RPT)
    prows = pl.ds(sid * _PD_T, _PD_T)
    pltpu.sync_copy(z128_hbm.at[pl.ds(0, _RPT)], accS.at[rows])

    @pl.when(sid < _PD_NT)
    def _zero_den():
        pltpu.sync_copy(z128_hbm.at[pl.ds(0, _PD_T)], denP.at[prows])
    if _REM:
        rem = pl.ds(_NS * _RPT, _REM)

        @pl.when(sid == _NS - 1)
        def _zero_rem():
            pltpu.sync_copy(z128_hbm.at[pl.ds(0, _REM)], accS.at[rem])

    pltpu.sync_copy(src_hbm.at[pl.ds(base, _IBK)], srcblk)
    pltpu.sync_copy(dst_hbm.at[pl.ds(base, _IBK)], dstblk)
    pltpu.async_copy(xl_hbm.at[srcblk.at[pl.ds(0, _K)]],
                     xlr.at[pl.ds(0, _K)], gsem.at[0])
    pltpu.async_copy(xr_hbm.at[dstblk.at[pl.ds(0, _K)]],
                     xrr.at[pl.ds(0, _K)], gsem.at[0])

    plsc.subcore_barrier()

    iota16 = lax.iota(jnp.int32, 16)
    zz = jnp.zeros((16,), jnp.float32)

    def _chunk(ci, carry):
        b = ci & 1
        j5 = ci // _IB
        ioff = (ci - j5 * _IB) * _K
        boff = b * _K

        pltpu.make_async_copy(
            xl_hbm.at[srcblk.at[pl.ds(ioff, _K)]],
            xlr.at[pl.ds(boff, _K)], gsem.at[b]).wait()
        pltpu.make_async_copy(
            xr_hbm.at[dstblk.at[pl.ds(ioff, _K)]],
            xrr.at[pl.ds(boff, _K)], gsem.at[b]).wait()

        @pl.when((ci > 0) & (b == 1))
        def _w2a():
            pltpu.make_async_copy(
                xlr.at[pl.ds(0, _K)], accS.at[dstv0], ssem.at[0]).wait()
            pltpu.make_async_copy(
                xrr.at[pl.ds(0, _K)], denP.at[dst8v0], ssem.at[0]).wait()

        @pl.when((ci > 0) & (b == 0))
        def _w2b():
            pltpu.make_async_copy(
                xlr.at[pl.ds(_K, _K)], accS.at[dstv1], ssem.at[1]).wait()
            pltpu.make_async_copy(
                xrr.at[pl.ds(_K, _K)], denP.at[dst8v1], ssem.at[1]).wait()

        bs16 = jnp.full((16,), boff, jnp.int32)
        is16 = jnp.full((16,), ioff, jnp.int32)
        m0 = bs16 == 0
        m1 = jnp.logical_not(m0)

        def _pre(g, c2):
            eids = iota16 + (g * 16)
            dv = plsc.load_gather(dstblk, [eids + is16])
            plsc.store_scatter(dstv0, [eids], dv, mask=m0)
            plsc.store_scatter(dstv1, [eids], dv, mask=m1)
            d8 = lax.shift_right_logical(dv, 5)
            plsc.store_scatter(dst8v0, [eids], d8, mask=m0)
            plsc.store_scatter(dst8v1, [eids], d8, mask=m1)
            dm = dv & 31
            plsc.store_scatter(cbv, [eids],
                               lax.shift_left(lax.shift_right_logical(dm, 2),
                                              4))
            plsc.store_scatter(wov, [eids], (dm & 3) * 4)
            return c2
        lax.fori_loop(0, _K // 16, _pre, 0)

        nb = ci + 1

        @pl.when((nb % _IB == 0) & (nb < _NCHUNK))
        def _i1():
            pltpu.sync_copy(src_hbm.at[pl.ds(base + nb * _K, _IBK)], srcblk)
            pltpu.sync_copy(dst_hbm.at[pl.ds(base + nb * _K, _IBK)], dstblk)

        @pl.when(nb < _NCHUNK)
        def _i2():
            j5n = nb // _IB
            ioffn = (nb - j5n * _IB) * _K
            bn = 1 - b
            pltpu.async_copy(xl_hbm.at[srcblk.at[pl.ds(ioffn, _K)]],
                             xlr.at[pl.ds(bn * _K, _K)], gsem.at[bn])
            pltpu.async_copy(xr_hbm.at[dstblk.at[pl.ds(ioffn, _K)]],
                             xrr.at[pl.ds(bn * _K, _K)], gsem.at[bn])


        att8 = [att_v[pl.ds(16 * bk, 16)] for bk in range(_D // 16)]

        def _edge(e, c2):
            row = boff + e
            xlv = [xlr[row, pl.ds(16 * bk, 16)] for bk in range(_D // 16)]
            w = []
            for bk in range(_D // 16):
                t = xlv[bk] + xrr[row, pl.ds(16 * bk, 16)]
                t = jnp.maximum(t, 0.2 * t)
                w.append(t * att8[bk])
            pv = []
            for h in range(_H):
                sh = w[2 * h] + w[2 * h + 1]
                r = jnp.sum(sh)
                pv.append(jnp.exp(jnp.full((16,), r, jnp.float32)))
            for bk in range(_D // 16):
                xlr[row, pl.ds(16 * bk, 16)] = xlv[bk] * pv[bk // 2]
            es = jnp.full((16,), e, jnp.int32)
            cb16 = plsc.load_gather(cbv, [es])[0]
            l4 = iota16 - plsc.load_gather(wov, [es])
            pd = jnp.where(l4 == 0, pv[0],
                           jnp.where(l4 == 1, pv[1],
                                     jnp.where(l4 == 2, pv[2],
                                               jnp.where(l4 == 3, pv[3],
                                                         zz))))
            for bk in range(_D // 16):
                xrr[row, pl.ds(16 * bk, 16)] = zz
            xrr[row, pl.ds(cb16, 16)] = pd
            return c2

        def _edge_pl(e):
            _edge(e, 0)
        plsc.parallel_loop(0, _K, 1, unroll=4)(_edge_pl)

        @pl.when(b == 0)
        def _s0():
            pltpu.async_copy(xlr.at[pl.ds(0, _K)], accS.at[dstv0],
                             ssem.at[0], add=True)
            pltpu.async_copy(xrr.at[pl.ds(0, _K)], denP.at[dst8v0],
                             ssem.at[0], add=True)

        @pl.when(b == 1)
        def _s1():
            pltpu.async_copy(xlr.at[pl.ds(_K, _K)], accS.at[dstv1],
                             ssem.at[1], add=True)
            pltpu.async_copy(xrr.at[pl.ds(_K, _K)], denP.at[dst8v1],
                             ssem.at[1], add=True)
        return carry
    lax.fori_loop(0, _NCHUNK, _chunk, 0)

    bL = (_NCHUNK - 1) & 1
    if bL == 0:
        pltpu.make_async_copy(xlr.at[pl.ds(0, _K)], accS.at[dstv0],
                              ssem.at[0]).wait()
        pltpu.make_async_copy(xrr.at[pl.ds(0, _K)], denP.at[dst8v0],
                              ssem.at[0]).wait()
    else:
        pltpu.make_async_copy(xlr.at[pl.ds(_K, _K)], accS.at[dstv1],
                              ssem.at[1]).wait()
        pltpu.make_async_copy(xrr.at[pl.ds(_K, _K)], denP.at[dst8v1],
                              ssem.at[1]).wait()

    plsc.subcore_barrier()
    pltpu.sync_copy(accS.at[rows], acc_out.at[cid, rows])

    @pl.when(sid < _PD_NT)
    def _copy_den():
        pltpu.sync_copy(denP.at[prows], den_out.at[cid, prows])
    if _REM:
        rem = pl.ds(_NS * _RPT, _REM)

        @pl.when(sid == _NS - 1)
        def _copy_rem():
            pltpu.sync_copy(accS.at[rem], acc_out.at[cid, rem])


_edge_kernel = functools.partial(
    pl.kernel,
    out_type=[
        jax.ShapeDtypeStruct((_NC, _N, _D), jnp.float32),
        jax.ShapeDtypeStruct((_NC, _NPDA, _D), jnp.float32),
    ],
    mesh=plsc.VectorSubcoreMesh(core_axis_name="c", subcore_axis_name="s"),
    compiler_params=pltpu.CompilerParams(needs_layout_passes=False),
    scratch_types=[
        pltpu.VMEM((_IBK,), jnp.int32),
        pltpu.VMEM((_IBK,), jnp.int32),
        pltpu.VMEM((_K,), jnp.int32),
        pltpu.VMEM((_K,), jnp.int32),
        pltpu.VMEM((_K,), jnp.int32),
        pltpu.VMEM((_K,), jnp.int32),
        pltpu.VMEM((_K,), jnp.int32),
        pltpu.VMEM((_K,), jnp.int32),
        pltpu.VMEM((2 * _K, _D), jnp.float32),
        pltpu.VMEM((2 * _K, _D), jnp.float32),
        pltpu.VMEM((_D,), jnp.float32),
        pltpu.VMEM_SHARED((_N, _D), jnp.float32),
        pltpu.VMEM_SHARED((_NPDA, _D), jnp.float32),
        pltpu.SemaphoreType.DMA((2,)),
        pltpu.SemaphoreType.DMA((2,)),
    ],
)(_edge_body)


def _post_body(x_ref, xl_ref, xr_ref, acc_ref, den_ref, attf_ref, bias_ref,
               gnw_ref, gnb_ref, gms_ref, out_ref):
    xv = x_ref[...]
    xl = xl_ref[...]
    xr = xr_ref[...]
    t = xl + xr
    t = jnp.maximum(t, 0.2 * t)
    w = t * attf_ref[...]
    ii = lax.broadcasted_iota(jnp.int32, (_D, _H), 0) // _C
    hh = lax.broadcasted_iota(jnp.int32, (_D, _H), 1)
    sel = (ii == hh).astype(jnp.float32)
    logit_s = jnp.dot(w, sel, preferred_element_type=jnp.float32)
    p_s = jnp.exp(logit_s)
    expand = jnp.dot(p_s, sel.T, preferred_element_type=jnp.float32)
    num = acc_ref[0] + acc_ref[1] + expand * xl
    den_e = jnp.dot(den_ref[0] + den_ref[1], sel.T,
                    preferred_element_type=jnp.float32) + expand
    h = num / (den_e + 1e-16) + bias_ref[...] + xv
    mean = jnp.mean(h, axis=0, keepdims=True)
    o = h - mean * gms_ref[...]
    var = jnp.mean(o * o, axis=0, keepdims=True)
    g = gnw_ref[...] * o / jnp.sqrt(var + 1e-5) + gnb_ref[...]
    out_ref[...] = 0.5 * g * (1.0 + lax.erf(g * np.float32(1.0 / np.sqrt(2.0))))


def kernel(x, edge_index, W_l, W_r, att, bias, gn_weight, gn_bias,
           gn_mean_scale):
    src = edge_index[0].astype(jnp.int32)
    dst = edge_index[1].astype(jnp.int32)
    attf = att.reshape(_H * _C).astype(jnp.float32)
    xl, xr = pl.pallas_call(
        _mm_body,
        out_shape=[jax.ShapeDtypeStruct((_N, _D), jnp.float32)] * 2,
    )(x, W_l, W_r)
    z128 = jnp.zeros((_RPT, _D), jnp.float32)
    acc, den_pack = _edge_kernel(xl, xr, src, dst, attf, z128)
    den4 = den_pack.reshape(_NC, _NPDA * 32, _H)[:, :_N]
    out = pl.pallas_call(
        _post_body,
        out_shape=jax.ShapeDtypeStruct((_N, _D), jnp.float32),
    )(x, xl, xr, acc, den4, attf.reshape(1, _D), bias.reshape(1, _D),
      gn_weight.reshape(1, _D), gn_bias.reshape(1, _D),
      gn_mean_scale.reshape(1, _D))
    return out
